# Initial kernel scaffold; baseline (speedup 1.0000x reference)
import jax
import jax.numpy as jnp
from jax.experimental import pallas as pl


def kernel(x, table):
    raise NotImplementedError("write your pallas kernel here")


# speedup vs baseline: 8.8650x; 8.8650x over previous
import functools

import jax
import jax.numpy as jnp
from jax import lax
from jax.experimental import pallas as pl
from jax.experimental.pallas import tpu as pltpu
from jax.experimental.pallas import tpu_sc as plsc

_B = 4096
_L = 200
_D = 64
_LANES = 16
_NVREG = _D // _LANES

_info = plsc.get_sparse_core_info()
_NC = _info.num_cores
_NS = _info.num_subcores
_NW = _NC * _NS
_NB = _B // _NW

_CHUNKS = ((0, 104), (104, 96))

_mesh = plsc.VectorSubcoreMesh(core_axis_name="c", subcore_axis_name="s")


@functools.partial(
    pl.kernel,
    mesh=_mesh,
    out_type=jax.ShapeDtypeStruct((_B, _D), jnp.float32),
    scratch_types=[
        pltpu.VMEM((_NB * _L,), jnp.int32),
        pltpu.VMEM((_L, _D), jnp.float32),
        pltpu.VMEM((_NB, _D), jnp.float32),
        pltpu.SemaphoreType.DMA,
    ],
    compiler_params=pltpu.CompilerParams(use_tc_tiling_on_sc=False),
)
def _encode(x_hbm, table_hbm, out_hbm, idx_v, rows_v, out_v, sem):
    wid = lax.axis_index("s") * _NC + lax.axis_index("c")
    base = pl.multiple_of(wid * _NB, _NB)

    pltpu.sync_copy(x_hbm.at[pl.ds(base * _L, _NB * _L)], idx_v)

    def per_element(b, carry):
        boff = pl.multiple_of(b * _L, 8)
        copies = [
            pltpu.async_copy(
                table_hbm.at[idx_v.at[pl.ds(boff + off, n)]],
                rows_v.at[pl.ds(off, n), :],
                sem,
            )
            for off, n in _CHUNKS
        ]
        for cp in copies:
            cp.wait()

        def reduce_row(l, acc):
            return tuple(
                acc[k] + rows_v[l, pl.ds(k * _LANES, _LANES)]
                for k in range(_NVREG)
            )

        acc = lax.fori_loop(
            0, _L, reduce_row,
            tuple(jnp.zeros((_LANES,), jnp.float32) for _ in range(_NVREG)),
        )
        scale = jnp.float32(1.0 / _L)
        for k in range(_NVREG):
            out_v[b, pl.ds(k * _LANES, _LANES)] = acc[k] * scale
        return carry

    lax.fori_loop(0, _NB, per_element, 0)

    pltpu.sync_copy(out_v, out_hbm.at[pl.ds(base, _NB), :])


def kernel(x, table):
    x_flat = x.astype(jnp.int32).reshape(_B * _L)
    return _encode(x_flat, table)


# speedup vs baseline: 16.0147x; 1.8065x over previous
import functools

import jax
import jax.numpy as jnp
from jax import lax
from jax.experimental import pallas as pl
from jax.experimental.pallas import tpu as pltpu
from jax.experimental.pallas import tpu_sc as plsc

_B = 4096
_L = 200
_D = 64
_LANES = 16
_NVREG = _D // _LANES

_info = plsc.get_sparse_core_info()
_NC = _info.num_cores
_NS = _info.num_subcores
_NW = _NC * _NS
_NB = _B // _NW
_NBUF = 4

_CHUNKS = ((0, 104), (104, 96))

_mesh = plsc.VectorSubcoreMesh(core_axis_name="c", subcore_axis_name="s")


@functools.partial(
    pl.kernel,
    mesh=_mesh,
    out_type=jax.ShapeDtypeStruct((_B, _D), jnp.float32),
    scratch_types=(
        [pltpu.VMEM((_NB * _L,), jnp.int32)]
        + [pltpu.VMEM((_L, _D), jnp.float32)] * _NBUF
        + [pltpu.VMEM((_NB, _D), jnp.float32)]
        + [pltpu.SemaphoreType.DMA] * _NBUF
    ),
    compiler_params=pltpu.CompilerParams(use_tc_tiling_on_sc=False),
)
def _encode(x_hbm, table_hbm, out_hbm, idx_v, *rest):
    rows = rest[:_NBUF]
    out_v = rest[_NBUF]
    sems = rest[_NBUF + 1:]

    wid = lax.axis_index("s") * _NC + lax.axis_index("c")
    base = pl.multiple_of(wid * _NB, _NB)

    pltpu.sync_copy(x_hbm.at[pl.ds(base * _L, _NB * _L)], idx_v)

    def fire(b, p):
        boff = pl.multiple_of(b * _L, 8)
        for off, n in _CHUNKS:
            pltpu.async_copy(
                table_hbm.at[idx_v.at[pl.ds(boff + off, n)]],
                rows[p].at[pl.ds(off, n), :],
                sems[p],
            )

    def drain(p):
        for off, n in _CHUNKS:
            pltpu.make_async_copy(
                table_hbm.at[idx_v.at[pl.ds(off, n)]],
                rows[p].at[pl.ds(off, n), :],
                sems[p],
            ).wait()

    def reduce_into(b, p):
        def reduce_row(l, acc):
            return tuple(
                acc[k] + rows[p][l, pl.ds(k * _LANES, _LANES)]
                for k in range(_NVREG)
            )

        acc = lax.fori_loop(
            0, _L, reduce_row,
            tuple(jnp.zeros((_LANES,), jnp.float32) for _ in range(_NVREG)),
        )
        scale = jnp.float32(1.0 / _L)
        for k in range(_NVREG):
            out_v[b, pl.ds(k * _LANES, _LANES)] = acc[k] * scale

    for p in range(_NBUF):
        fire(jnp.int32(p), p)

    def group(i, carry):
        b0 = i * _NBUF
        for p in range(_NBUF):
            b = b0 + p
            drain(p)
            reduce_into(b, p)

            @pl.when(b + _NBUF < _NB)
            def _():
                fire(b + _NBUF, p)

        return carry

    lax.fori_loop(0, _NB // _NBUF, group, 0)

    pltpu.sync_copy(out_v, out_hbm.at[pl.ds(base, _NB), :])


def kernel(x, table):
    x_flat = x.astype(jnp.int32).reshape(_B * _L)
    return _encode(x_flat, table)


# speedup vs baseline: 18.1423x; 1.1329x over previous
import functools

import jax
import jax.numpy as jnp
from jax import lax
from jax.experimental import pallas as pl
from jax.experimental.pallas import tpu as pltpu
from jax.experimental.pallas import tpu_sc as plsc

_B = 4096
_L = 200
_D = 64
_LANES = 16
_NVREG = _D // _LANES

_info = plsc.get_sparse_core_info()
_NC = _info.num_cores
_NS = _info.num_subcores
_NW = _NC * _NS
_NB = _B // _NW
_NBUF = 4

_CHUNKS = ((0, 104), (104, 96))

_mesh = plsc.VectorSubcoreMesh(core_axis_name="c", subcore_axis_name="s")


@functools.partial(
    pl.kernel,
    mesh=_mesh,
    out_type=jax.ShapeDtypeStruct((_B, _D), jnp.float32),
    scratch_types=(
        [pltpu.VMEM((_NB * _L,), jnp.int32)]
        + [pltpu.VMEM((_L, _D), jnp.float32)] * _NBUF
        + [pltpu.VMEM((_NB, _D), jnp.float32)]
        + [pltpu.SemaphoreType.DMA] * _NBUF
    ),
    compiler_params=pltpu.CompilerParams(use_tc_tiling_on_sc=False),
)
def _encode(x_hbm, table_hbm, out_hbm, idx_v, *rest):
    rows = rest[:_NBUF]
    out_v = rest[_NBUF]
    sems = rest[_NBUF + 1:]

    wid = lax.axis_index("s") * _NC + lax.axis_index("c")
    base = pl.multiple_of(wid * _NB, _NB)

    pltpu.sync_copy(x_hbm.at[pl.ds(base * _L, _NB * _L)], idx_v)

    def fire(b, p):
        boff = pl.multiple_of(b * _L, 8)
        for off, n in _CHUNKS:
            pltpu.async_copy(
                table_hbm.at[idx_v.at[pl.ds(boff + off, n)]],
                rows[p].at[pl.ds(off, n), :],
                sems[p],
            )

    def drain(p):
        for off, n in _CHUNKS:
            pltpu.make_async_copy(
                table_hbm.at[idx_v.at[pl.ds(off, n)]],
                rows[p].at[pl.ds(off, n), :],
                sems[p],
            ).wait()

    def reduce_into(b, p):
        def reduce_rows8(i, acc):
            l0 = i * 8
            for j in range(8):
                acc = tuple(
                    acc[k] + rows[p][l0 + j, pl.ds(k * _LANES, _LANES)]
                    for k in range(_NVREG)
                )
            return acc

        acc = lax.fori_loop(
            0, _L // 8, reduce_rows8,
            tuple(jnp.zeros((_LANES,), jnp.float32) for _ in range(_NVREG)),
        )
        scale = jnp.float32(1.0 / _L)
        for k in range(_NVREG):
            out_v[b, pl.ds(k * _LANES, _LANES)] = acc[k] * scale

    for p in range(_NBUF):
        fire(jnp.int32(p), p)

    def group(i, carry):
        b0 = i * _NBUF
        for p in range(_NBUF):
            b = b0 + p
            drain(p)
            reduce_into(b, p)

            @pl.when(b + _NBUF < _NB)
            def _():
                fire(b + _NBUF, p)

        return carry

    lax.fori_loop(0, _NB // _NBUF, group, 0)

    pltpu.sync_copy(out_v, out_hbm.at[pl.ds(base, _NB), :])


def kernel(x, table):
    x_flat = x.astype(jnp.int32).reshape(_B * _L)
    return _encode(x_flat, table)
